# baseline (speedup 1.0000x reference)
import functools
import jax
import jax.numpy as jnp
from jax import lax
from jax.experimental import pallas as pl
from jax.experimental.pallas import tpu as pltpu
from jax.experimental.pallas import tpu_sc as plsc

_BONE_A = (0, 0, 1, 2, 5, 5, 7, 6, 8, 5, 6, 11, 11, 13, 12, 14)
_BONE_B = (1, 2, 3, 4, 6, 7, 9, 8, 10, 11, 12, 12, 13, 15, 14, 16)
_B, _NP, _NJ, _ND, _NB = 16, 20, 17, 64, 16
_NJP = 24
_L = 16
_NG = _ND // _L
_IMW, _IMH = 1920.0, 1080.0


def _spl(s):
    return jnp.full((_L,), s, dtype=jnp.float32)


def _sqrt16(x):
    i = lax.bitcast_convert_type(x, jnp.int32)
    i = jnp.int32(0x5F3759DF) - (i >> 1)
    y = lax.bitcast_convert_type(i, jnp.float32)
    xh = x * jnp.float32(0.5)
    for _ in range(3):
        y = y * (jnp.float32(1.5) - xh * y * y)
    return x * y


def _sc_body(cams_hbm, x3_hbm, y3_hbm, z3_hbm, xr_hbm, yr_hbm, vis_hbm,
             score_hbm, sbl_hbm, bound_hbm, bound2_hbm,
             cams_v, xr_v, yr_v, vis_v, x3_v, y3_v, z3_v,
             xt_v, yt_v, mx_v, my_v,
             score_v, sbl_v, bound_v, bound2_v, sem):
    nc = 2
    wid = lax.axis_index("s") * nc + lax.axis_index("c")
    b = wid // 2
    pt0 = (wid % 2) * (_NP // 2)

    c1 = pltpu.async_copy(cams_hbm.at[b], cams_v, sem)
    c2 = pltpu.async_copy(xr_hbm.at[b], xr_v, sem)
    c3 = pltpu.async_copy(yr_hbm.at[b], yr_v, sem)
    c4 = pltpu.async_copy(vis_hbm.at[b], vis_v, sem)
    c1.wait(); c2.wait(); c3.wait(); c4.wait()

    fxv = cams_v[0]
    fyv = cams_v[1]
    cxv = cams_v[2]
    cyv = cams_v[3]
    nprv = cams_v[4]

    def pt_body(pti, _):
        pt = pt0 + pti
        d1 = pltpu.async_copy(x3_hbm.at[b, pt], x3_v, sem)
        d2 = pltpu.async_copy(y3_hbm.at[b, pt], y3_v, sem)
        d3 = pltpu.async_copy(z3_hbm.at[b, pt], z3_v, sem)
        d1.wait(); d2.wait(); d3.wait()

        def proj(j, _):
            for g in range(_NG):
                sl = pl.ds(g * _L, _L)
                z = jnp.maximum(z3_v[j, sl], jnp.float32(1e-3))
                xt_v[j, sl] = x3_v[j, sl] / z * fxv + cxv
                yt_v[j, sl] = y3_v[j, sl] / z * fyv + cyv
            return 0
        lax.fori_loop(0, _NJ, proj, 0, unroll=False)

        def g_body(g, _):
            sl = pl.ds(g * _L, _L)

            def pr_body(pr, carry):
                bd, bi = carry
                pr16 = jnp.full((_L,), pr, jnp.int32)
                base = pr16 * _NJP
                acc = jnp.zeros((_L,), jnp.float32)
                den = _spl(jnp.float32(1e-8))
                for j in range(_NJ):
                    idxj = base + j
                    xrs = plsc.load_gather(xr_v, [idxj])
                    yrs = plsc.load_gather(yr_v, [idxj])
                    vs = plsc.load_gather(vis_v, [idxj])
                    dx = xt_v[j, sl] - xrs
                    dy = yt_v[j, sl] - yrs
                    acc = acc + vs * (dx * dx + dy * dy)
                    den = den + vs
                d = acc / den
                d = jnp.where(pr16.astype(jnp.float32) < nprv, d,
                              _spl(jnp.float32(1e5)))
                take = d < bd
                bi = jnp.where(take, pr16, bi)
                bd = jnp.where(take, d, bd)
                return bd, bi

            bd0 = jnp.full((_L,), jnp.float32(3.0e38))
            bi0 = jnp.zeros((_L,), jnp.int32)
            _, bi = lax.fori_loop(0, _NP, pr_body, (bd0, bi0), unroll=False)

            bibase = bi * _NJP
            mv0 = None
            for j in range(_NJ):
                idxj = bibase + j
                mx = plsc.load_gather(xr_v, [idxj])
                my = plsc.load_gather(yr_v, [idxj])
                mv = plsc.load_gather(vis_v, [idxj])
                mx_v[j] = mx
                my_v[j] = my
                xtj = xt_v[j, sl]
                ytj = yt_v[j, sl]
                ddx = xtj - mx
                ddy = ytj - my
                md = ddx * ddx + ddy * ddy + jnp.float32(1e-12)
                score_v[j, sl] = jnp.exp(_sqrt16(md) * jnp.float32(-1.0 / 50.0))
                inb = ((xtj >= jnp.float32(0.0)) & (ytj >= jnp.float32(0.0))
                       & (xtj <= jnp.float32(_IMW - 1))
                       & (ytj <= jnp.float32(_IMH - 1)))
                bound_v[j, sl] = jnp.where(inb, mv, _spl(jnp.float32(0.0)))
                if j == 0:
                    mv0 = mv

            for k in range(_NB):
                a, c = _BONE_A[k], _BONE_B[k]
                ex = xt_v[a, sl] - xt_v[c, sl]
                ey = yt_v[a, sl] - yt_v[c, sl]
                blt = _sqrt16(ex * ex + ey * ey + jnp.float32(1e-12))
                exr = mx_v[a] - mx_v[c]
                eyr = my_v[a] - my_v[c]
                blr = _sqrt16(exr * exr + eyr * eyr + jnp.float32(1e-12))
                sbl_v[k, sl] = jnp.exp(jnp.abs(blr - blt) * jnp.float32(-1.0 / 5.0))
                bound2_v[k, sl] = mv0
            return 0

        lax.fori_loop(0, _NG, g_body, 0, unroll=False)

        o1 = pltpu.async_copy(score_v, score_hbm.at[b, pt], sem)
        o2 = pltpu.async_copy(sbl_v, sbl_hbm.at[b, pt], sem)
        o3 = pltpu.async_copy(bound_v, bound_hbm.at[b, pt], sem)
        o4 = pltpu.async_copy(bound2_v, bound2_hbm.at[b, pt], sem)
        o1.wait(); o2.wait(); o3.wait(); o4.wait()
        return 0

    lax.fori_loop(0, _NP // 2, pt_body, 0, unroll=False)


@jax.jit
def kernel(poses_3d, poses_2d_ref, vis_ref, cam_f, cam_c, num_persons_ref):
    f32 = jnp.float32
    x3 = poses_3d[..., 0]
    y3 = poses_3d[..., 1]
    z3 = poses_3d[..., 2]
    pad = ((0, 0), (0, 0), (0, _NJP - _NJ))
    xr = jnp.pad(poses_2d_ref[..., 0], pad).reshape(_B, _NP * _NJP)
    yr = jnp.pad(poses_2d_ref[..., 1], pad).reshape(_B, _NP * _NJP)
    vis = jnp.pad(vis_ref, pad).reshape(_B, _NP * _NJP)
    cams = jnp.stack([cam_f[:, 0], cam_f[:, 1], cam_c[:, 0], cam_c[:, 1],
                      num_persons_ref.astype(f32),
                      jnp.zeros((_B,), f32), jnp.zeros((_B,), f32),
                      jnp.zeros((_B,), f32)], axis=1)
    cams16 = jnp.broadcast_to(cams[:, :, None], (_B, 8, _L)) + 0.0

    mesh = plsc.VectorSubcoreMesh(core_axis_name="c", subcore_axis_name="s",
                                  num_cores=2, num_subcores=16)
    out_type = [
        jax.ShapeDtypeStruct((_B, _NP, _NJ, _ND), f32),
        jax.ShapeDtypeStruct((_B, _NP, _NB, _ND), f32),
        jax.ShapeDtypeStruct((_B, _NP, _NJ, _ND), f32),
        jax.ShapeDtypeStruct((_B, _NP, _NB, _ND), f32),
    ]
    scratch = [
        pltpu.VMEM((8, _L), f32),
        pltpu.VMEM((_NP * _NJP,), f32),
        pltpu.VMEM((_NP * _NJP,), f32),
        pltpu.VMEM((_NP * _NJP,), f32),
        pltpu.VMEM((_NJ, _ND), f32),
        pltpu.VMEM((_NJ, _ND), f32),
        pltpu.VMEM((_NJ, _ND), f32),
        pltpu.VMEM((_NJ, _ND), f32),
        pltpu.VMEM((_NJ, _ND), f32),
        pltpu.VMEM((_NJ, _L), f32),
        pltpu.VMEM((_NJ, _L), f32),
        pltpu.VMEM((_NJ, _ND), f32),
        pltpu.VMEM((_NB, _ND), f32),
        pltpu.VMEM((_NJ, _ND), f32),
        pltpu.VMEM((_NB, _ND), f32),
        pltpu.SemaphoreType.DMA,
    ]
    outs = pl.kernel(
        _sc_body,
        out_type=out_type,
        mesh=mesh,
        scratch_types=scratch,
        compiler_params=pltpu.CompilerParams(needs_layout_passes=False),
    )(cams16, x3, y3, z3, xr, yr, vis)
    return tuple(outs)
